# baseline (speedup 1.0000x reference)
import jax
import jax.numpy as jnp
from jax import lax
from jax.experimental import pallas as pl
from jax.experimental.pallas import tpu as pltpu

_BM = 512


def _router_body(h_ref, w_ref, b_ref, logits_ref, probs_ref, depth_ref):
    h = h_ref[...]
    w = w_ref[...]
    b = b_ref[...]
    logits = jnp.dot(h, w, preferred_element_type=jnp.float32) + b
    logits_ref[...] = logits
    m = jnp.max(logits, axis=-1, keepdims=True)
    e = jnp.exp(logits - m)
    s = jnp.sum(e, axis=-1, keepdims=True)
    probs_ref[...] = e / s
    c = logits.shape[-1]
    iota = lax.broadcasted_iota(jnp.int32, logits.shape, 1)
    cand = jnp.where(logits == m, iota, c)
    idx = jnp.min(cand, axis=-1)
    depth_ref[...] = (idx + 1)[:, None]


def kernel(hidden_states, W, b):
    B, S, D = hidden_states.shape
    C = W.shape[-1]
    N = B * S
    h2 = hidden_states.reshape(N, D)
    b2 = b.reshape(1, C)

    grid = (N // _BM,)
    logits, probs, depth = pl.pallas_call(
        _router_body,
        grid=grid,
        in_specs=[
            pl.BlockSpec((_BM, D), lambda i: (i, 0)),
            pl.BlockSpec((D, C), lambda i: (0, 0)),
            pl.BlockSpec((1, C), lambda i: (0, 0)),
        ],
        out_specs=[
            pl.BlockSpec((_BM, C), lambda i: (i, 0)),
            pl.BlockSpec((_BM, C), lambda i: (i, 0)),
            pl.BlockSpec((_BM, 1), lambda i: (i, 0)),
        ],
        out_shape=[
            jax.ShapeDtypeStruct((N, C), jnp.float32),
            jax.ShapeDtypeStruct((N, C), jnp.float32),
            jax.ShapeDtypeStruct((N, 1), jnp.int32),
        ],
        compiler_params=pltpu.CompilerParams(
            dimension_semantics=("arbitrary",),
        ),
    )(h2, W, b2)

    depth_values = depth.reshape(B, S)
    last_loss = jnp.zeros((), dtype=jnp.float32)
    return (depth_values, probs.reshape(B, S, C), logits.reshape(B, S, C),
            last_loss)


# speedup vs baseline: 1.1003x; 1.1003x over previous
import jax
import jax.numpy as jnp
from jax import lax
from jax.experimental import pallas as pl
from jax.experimental.pallas import tpu as pltpu

_BM = 1024


def _router_body(h_ref, w_ref, b_ref, logits_ref, probs_ref, depth_ref):
    h = h_ref[...]
    w = w_ref[...]
    b = b_ref[...]
    logits = jnp.dot(h, w, preferred_element_type=jnp.float32) + b
    logits_ref[...] = logits
    m = jnp.max(logits, axis=-1, keepdims=True)
    e = jnp.exp(logits - m)
    s = jnp.sum(e, axis=-1, keepdims=True)
    probs_ref[...] = e / s
    c = logits.shape[-1]
    iota = lax.broadcasted_iota(jnp.int32, logits.shape, 1)
    cand = jnp.where(logits == m, iota, c)
    idx = jnp.min(cand, axis=-1)
    depth_ref[...] = (idx + 1)[:, None]


def kernel(hidden_states, W, b):
    B, S, D = hidden_states.shape
    C = W.shape[-1]
    N = B * S
    h2 = hidden_states.reshape(N, D)
    b2 = b.reshape(1, C)

    grid = (N // _BM,)
    logits, probs, depth = pl.pallas_call(
        _router_body,
        grid=grid,
        in_specs=[
            pl.BlockSpec((_BM, D), lambda i: (i, 0)),
            pl.BlockSpec((D, C), lambda i: (0, 0)),
            pl.BlockSpec((1, C), lambda i: (0, 0)),
        ],
        out_specs=[
            pl.BlockSpec((_BM, C), lambda i: (i, 0)),
            pl.BlockSpec((_BM, C), lambda i: (i, 0)),
            pl.BlockSpec((_BM, 1), lambda i: (i, 0)),
        ],
        out_shape=[
            jax.ShapeDtypeStruct((N, C), jnp.float32),
            jax.ShapeDtypeStruct((N, C), jnp.float32),
            jax.ShapeDtypeStruct((N, 1), jnp.int32),
        ],
        compiler_params=pltpu.CompilerParams(
            dimension_semantics=("arbitrary",),
        ),
    )(h2, W, b2)

    depth_values = depth.reshape(B, S)
    last_loss = jnp.zeros((), dtype=jnp.float32)
    return (depth_values, probs.reshape(B, S, C), logits.reshape(B, S, C),
            last_loss)


# speedup vs baseline: 1.1119x; 1.0105x over previous
import jax
import jax.numpy as jnp
from jax import lax
from jax.experimental import pallas as pl
from jax.experimental.pallas import tpu as pltpu

_CHUNK = 512
_NBUF = 6


def _router_body(h_hbm, w_ref, b_ref, logits_ref, probs_ref, depth_ref,
                 h_buf, sems):
    i = pl.program_id(0)
    n = pl.num_programs(0)

    def start_copy(chunk_idx, buf_idx):
        pltpu.make_async_copy(
            h_hbm.at[pl.ds(chunk_idx * _CHUNK, _CHUNK), :],
            h_buf.at[buf_idx],
            sems.at[buf_idx],
        ).start()

    @pl.when(i == 0)
    def _prologue():
        for k in range(_NBUF):
            start_copy(k, k)

    buf = lax.rem(i, _NBUF)
    pltpu.make_async_copy(
        h_hbm.at[pl.ds(i * _CHUNK, _CHUNK), :],
        h_buf.at[buf],
        sems.at[buf],
    ).wait()

    h = h_buf[buf]
    w = w_ref[...]
    b = b_ref[...]
    logits = jnp.dot(h, w, preferred_element_type=jnp.float32) + b
    logits_ref[...] = logits
    m = jnp.max(logits, axis=-1, keepdims=True)
    e = jnp.exp(logits - m)
    s = jnp.sum(e, axis=-1, keepdims=True)
    probs_ref[...] = e / s
    c = logits.shape[-1]
    iota = lax.broadcasted_iota(jnp.int32, logits.shape, 1)
    cand = jnp.where(logits == m, iota, c)
    idx = jnp.min(cand, axis=-1)
    depth_ref[...] = (idx + 1)[:, None]

    @pl.when(i + _NBUF < n)
    def _prefetch():
        start_copy(i + _NBUF, buf)


def kernel(hidden_states, W, b):
    B, S, D = hidden_states.shape
    C = W.shape[-1]
    N = B * S
    h2 = hidden_states.reshape(N, D)
    b2 = b.reshape(1, C)

    grid = (N // _CHUNK,)
    logits, probs, depth = pl.pallas_call(
        _router_body,
        grid=grid,
        in_specs=[
            pl.BlockSpec(memory_space=pl.ANY),
            pl.BlockSpec((D, C), lambda i: (0, 0)),
            pl.BlockSpec((1, C), lambda i: (0, 0)),
        ],
        out_specs=[
            pl.BlockSpec((_CHUNK, C), lambda i: (i, 0)),
            pl.BlockSpec((_CHUNK, C), lambda i: (i, 0)),
            pl.BlockSpec((_CHUNK, 1), lambda i: (i, 0)),
        ],
        out_shape=[
            jax.ShapeDtypeStruct((N, C), jnp.float32),
            jax.ShapeDtypeStruct((N, C), jnp.float32),
            jax.ShapeDtypeStruct((N, 1), jnp.int32),
        ],
        scratch_shapes=[
            pltpu.VMEM((_NBUF, _CHUNK, D), jnp.float32),
            pltpu.SemaphoreType.DMA((_NBUF,)),
        ],
        compiler_params=pltpu.CompilerParams(
            dimension_semantics=("arbitrary",),
        ),
    )(h2, W, b2)

    depth_values = depth.reshape(B, S)
    last_loss = jnp.zeros((), dtype=jnp.float32)
    return (depth_values, probs.reshape(B, S, C), logits.reshape(B, S, C),
            last_loss)


# speedup vs baseline: 1.5996x; 1.4386x over previous
import jax
import jax.numpy as jnp
from jax import lax
from jax.experimental import pallas as pl
from jax.experimental.pallas import tpu as pltpu

_BM = 1024


def _router_body(h_ref, w_ref, bt_ref, logits_ref, probs_ref, depth_ref):
    h = h_ref[...]
    w = w_ref[...]
    bt = bt_ref[...]
    logits = lax.dot_general(w, h, (((0,), (1,)), ((), ())),
                             preferred_element_type=jnp.float32) + bt
    logits_ref[...] = logits
    m = jnp.max(logits, axis=0, keepdims=True)
    e = jnp.exp(logits - m)
    s = jnp.sum(e, axis=0, keepdims=True)
    probs_ref[...] = e / s
    c = logits.shape[0]
    iota = lax.broadcasted_iota(jnp.int32, logits.shape, 0)
    cand = jnp.where(logits == m, iota, c)
    idx = jnp.min(cand, axis=0, keepdims=True)
    depth_ref[...] = idx + 1


def kernel(hidden_states, W, b):
    B, S, D = hidden_states.shape
    C = W.shape[-1]
    N = B * S
    h2 = hidden_states.reshape(N, D)
    bt = b.reshape(C, 1)

    grid = (N // _BM,)
    logitsT, probsT, depthT = pl.pallas_call(
        _router_body,
        grid=grid,
        in_specs=[
            pl.BlockSpec((_BM, D), lambda i: (i, 0)),
            pl.BlockSpec((D, C), lambda i: (0, 0)),
            pl.BlockSpec((C, 1), lambda i: (0, 0)),
        ],
        out_specs=[
            pl.BlockSpec((C, _BM), lambda i: (0, i)),
            pl.BlockSpec((C, _BM), lambda i: (0, i)),
            pl.BlockSpec((1, _BM), lambda i: (0, i)),
        ],
        out_shape=[
            jax.ShapeDtypeStruct((C, N), jnp.float32),
            jax.ShapeDtypeStruct((C, N), jnp.float32),
            jax.ShapeDtypeStruct((1, N), jnp.int32),
        ],
        compiler_params=pltpu.CompilerParams(
            dimension_semantics=("arbitrary",),
        ),
    )(h2, W, bt)

    depth_values = depthT.reshape(B, S)
    last_loss = jnp.zeros((), dtype=jnp.float32)
    return (depth_values, probsT.T.reshape(B, S, C),
            logitsT.T.reshape(B, S, C), last_loss)
